# baseline (speedup 1.0000x reference)
import functools

import jax
import jax.numpy as jnp
from jax import lax
from jax.experimental import pallas as pl
from jax.experimental.pallas import tpu as pltpu
from jax.experimental.pallas import tpu_sc as plsc

B = 1024
M = 100000
L = 16384

NC = 2
NS = 16
NW = NC * NS
ROWS_PER_W = B // NW
XCH = 8192
NCH = L // XCH


def _scatter_body(out_hbm, x0_hbm, ind_hbm, res_hbm, row_v, ind_v, x0_v):
    wid = lax.axis_index("s") * NC + lax.axis_index("c")
    pltpu.sync_copy(ind_hbm, ind_v)

    def row_loop(k, carry):
        b = wid * ROWS_PER_W + k
        pltpu.sync_copy(out_hbm.at[b], row_v)

        def chunk_loop(c, carry2):
            pltpu.sync_copy(x0_hbm.at[b, pl.ds(c * XCH, XCH)], x0_v)

            def vec_loop(i, carry3):
                idx = ind_v[pl.ds(c * XCH + i * 16, 16)]
                x = x0_v[pl.ds(i * 16, 16)]
                plsc.addupdate_scatter(row_v, [idx], x)
                return carry3

            return lax.fori_loop(0, XCH // 16, vec_loop, carry2)

        lax.fori_loop(0, NCH, chunk_loop, carry)
        pltpu.sync_copy(row_v, res_hbm.at[b])
        return carry

    lax.fori_loop(0, ROWS_PER_W, row_loop, 0)


def kernel(out, x0, ind1):
    mesh = plsc.VectorSubcoreMesh(core_axis_name="c", subcore_axis_name="s")
    k = pl.kernel(
        _scatter_body,
        out_type=jax.ShapeDtypeStruct((B, M), jnp.float32),
        mesh=mesh,
        scratch_types=[
            pltpu.VMEM((M,), jnp.float32),
            pltpu.VMEM((L,), jnp.int32),
            pltpu.VMEM((XCH,), jnp.float32),
        ],
        compiler_params=pltpu.CompilerParams(needs_layout_passes=False),
    )
    return k(out, x0, ind1)
